# Initial kernel scaffold; baseline (speedup 1.0000x reference)
import jax
import jax.numpy as jnp
from jax.experimental import pallas as pl


def kernel(x, w_t, gamma, beta):
    raise NotImplementedError("write your pallas kernel here")


# speedup vs baseline: 1.3729x; 1.3729x over previous
import functools

import jax
import jax.numpy as jnp
from jax import lax
from jax.experimental import pallas as pl
from jax.experimental.pallas import tpu as pltpu

_IF = 4
_OF = 8
_EPS = 1e-5


def _shift_stack(x2, D, H, W):
    P = D * H * W
    idx = lax.broadcasted_iota(jnp.int32, (_IF, P), 1)
    mask_w = (idx % W) < (W - 1)
    mask_h = ((idx // W) % H) < (H - 1)
    mask_d = (idx // (H * W)) < (D - 1)

    def shift(v, amt, mask):
        return jnp.where(mask, pltpu.roll(v, P - amt, axis=1), 0.0)

    rows = []
    for od in (0, 1):
        xd = shift(x2, H * W, mask_d) if od else x2
        for oh in (0, 1):
            xh = shift(xd, W, mask_h) if oh else xd
            for ow in (0, 1):
                rows.append(shift(xh, 1, mask_w) if ow else xh)
    return jnp.concatenate(rows, axis=0)


def _stats_kernel(x_ref, w_ref, s_ref, q_ref, *, D, H, W):
    xs = _shift_stack(x_ref[0], D, H, W)
    y = jnp.dot(w_ref[...], xs, preferred_element_type=jnp.float32)
    s_ref[0] = jnp.sum(y, axis=1, keepdims=True)
    q_ref[0] = jnp.sum(y * y, axis=1, keepdims=True)


def _fused_kernel(x_ref, w_ref, scale_ref, shift_ref, o_ref, *, D, H, W):
    xs = _shift_stack(x_ref[0], D, H, W)
    y = jnp.dot(w_ref[...], xs, preferred_element_type=jnp.float32)
    o_ref[0] = jnp.maximum(y * scale_ref[...] + shift_ref[...], 0.0)


def _fold_weights(w_t):
    t = jnp.array([[[0, 1, 0], [0, 0, 0]],
                   [[0, 0, 1], [1, 0, 0]]], w_t.dtype)
    w8 = jnp.einsum('ifxyz,pax,qby,rcz->pqfrabci', w_t, t, t, t)
    return w8.reshape(2 * 2 * _OF * 2, 8 * _IF)


def kernel(x, w_t, gamma, beta):
    N, C, D, H, W = x.shape
    P = D * H * W
    x2 = x.reshape(N, C, P)
    w_all = _fold_weights(w_t)

    s_p, q_p = pl.pallas_call(
        functools.partial(_stats_kernel, D=D, H=H, W=W),
        out_shape=(jax.ShapeDtypeStruct((N, 64, 1), jnp.float32),
                   jax.ShapeDtypeStruct((N, 64, 1), jnp.float32)),
        grid=(N,),
        in_specs=[
            pl.BlockSpec((1, C, P), lambda n: (n, 0, 0)),
            pl.BlockSpec((64, 32), lambda n: (0, 0)),
        ],
        out_specs=(
            pl.BlockSpec((1, 64, 1), lambda n: (n, 0, 0)),
            pl.BlockSpec((1, 64, 1), lambda n: (n, 0, 0)),
        ),
        compiler_params=pltpu.CompilerParams(
            dimension_semantics=("parallel",)),
    )(x2, w_all)

    count = float(N) * 8.0 * float(P)
    s_c = jnp.sum(s_p.reshape(N, 2, 2, _OF, 2), axis=(0, 1, 2, 4))
    q_c = jnp.sum(q_p.reshape(N, 2, 2, _OF, 2), axis=(0, 1, 2, 4))
    mean = s_c / count
    var = q_c / count - mean * mean
    scale = gamma * lax.rsqrt(var + _EPS)
    shift = beta - mean * scale
    scale_rows = jnp.broadcast_to(scale.reshape(1, 1, _OF, 1), (2, 2, _OF, 2)).reshape(64, 1)
    shift_rows = jnp.broadcast_to(shift.reshape(1, 1, _OF, 1), (2, 2, _OF, 2)).reshape(64, 1)

    out_cls = pl.pallas_call(
        functools.partial(_fused_kernel, D=D, H=H, W=W),
        out_shape=jax.ShapeDtypeStruct((N, 64, P), jnp.float32),
        grid=(N,),
        in_specs=[
            pl.BlockSpec((1, C, P), lambda n: (n, 0, 0)),
            pl.BlockSpec((64, 32), lambda n: (0, 0)),
            pl.BlockSpec((64, 1), lambda n: (0, 0)),
            pl.BlockSpec((64, 1), lambda n: (0, 0)),
        ],
        out_specs=pl.BlockSpec((1, 64, P), lambda n: (n, 0, 0)),
        compiler_params=pltpu.CompilerParams(
            dimension_semantics=("parallel",)),
    )(x2, w_all, scale_rows, shift_rows)

    o7 = out_cls.reshape(N, 2, 2, _OF, 2, D, H, W)
    out = o7.transpose(0, 3, 5, 1, 6, 2, 7, 4)
    return out.reshape(N, _OF, 2 * D, 2 * H, 2 * W)


# speedup vs baseline: 2.0544x; 1.4963x over previous
import functools

import jax
import jax.numpy as jnp
from jax import lax
from jax.experimental import pallas as pl
from jax.experimental.pallas import tpu as pltpu

_IF = 4
_OF = 8
_EPS = 1e-5


def _shift_stack(x2, D, H, W):
    P = D * H * W
    idx = lax.broadcasted_iota(jnp.int32, (_IF, P), 1)
    mask_w = (idx % W) < (W - 1)
    mask_h = ((idx // W) % H) < (H - 1)
    mask_d = (idx // (H * W)) < (D - 1)

    def shift(v, amt, mask):
        return jnp.where(mask, pltpu.roll(v, P - amt, axis=1), 0.0)

    rows = []
    for od in (0, 1):
        xd = shift(x2, H * W, mask_d) if od else x2
        for oh in (0, 1):
            xh = shift(xd, W, mask_h) if oh else xd
            for ow in (0, 1):
                rows.append(shift(xh, 1, mask_w) if ow else xh)
    return jnp.concatenate(rows, axis=0)


def _stats_kernel(x_ref, w_ref, s_ref, q_ref, *, D, H, W):
    xs = _shift_stack(x_ref[0], D, H, W)
    y = jnp.dot(w_ref[...], xs, preferred_element_type=jnp.float32)
    s_ref[0] = jnp.sum(y, axis=1, keepdims=True)
    q_ref[0] = jnp.sum(y * y, axis=1, keepdims=True)


def _fused_kernel(x_ref, w0_ref, w1_ref, scale_ref, shift_ref, o_ref, *, D, H, W):
    P2 = 2 * D * H * W
    xu = x_ref[0]
    idx = lax.broadcasted_iota(jnp.int32, (8 * _IF, P2), 1)
    q = idx // 2
    mask_w = (q % W) < (W - 1)
    mask_h = ((q // W) % H) < (H - 1)
    mask_d = (q // (H * W)) < (D - 1)

    def shift(v, amt, mask):
        r = v.shape[0]
        return jnp.where(mask[:r], pltpu.roll(v, P2 - 2 * amt, axis=1), 0.0)

    rows = []
    for od in (0, 1):
        xd = shift(xu, H * W, mask_d) if od else xu
        for oh in (0, 1):
            xh = shift(xd, W, mask_h) if oh else xd
            for ow in (0, 1):
                rows.append(shift(xh, 1, mask_w) if ow else xh)
    xs = jnp.concatenate(rows, axis=0)
    even = (idx % 2) == 0
    y = (jnp.dot(w0_ref[...], jnp.where(even, xs, 0.0),
                 preferred_element_type=jnp.float32) +
         jnp.dot(w1_ref[...], jnp.where(even, 0.0, xs),
                 preferred_element_type=jnp.float32))
    y = jnp.maximum(y * scale_ref[...] + shift_ref[...], 0.0)
    for pd in (0, 1):
        for ph in (0, 1):
            base = (pd * 2 + ph) * _OF
            u = y[base:base + _OF].reshape(_OF, D, H, 2 * W)
            o_ref[0, :, :, pd, :, ph, :] = u


def _fold_weights(w_t):
    t = jnp.array([[[0, 1, 0], [0, 0, 0]],
                   [[0, 0, 1], [1, 0, 0]]], w_t.dtype)
    w8 = jnp.einsum('ifxyz,pax,qby,rcz->rpqfabci', w_t, t, t, t)
    return w8.reshape(2 * 2 * _OF * 2, 8 * _IF)


def kernel(x, w_t, gamma, beta):
    N, C, D, H, W = x.shape
    P = D * H * W
    x2 = x.reshape(N, C, P)
    w_all = _fold_weights(w_t)

    s_p, q_p = pl.pallas_call(
        functools.partial(_stats_kernel, D=D, H=H, W=W),
        out_shape=(jax.ShapeDtypeStruct((N, 64, 1), jnp.float32),
                   jax.ShapeDtypeStruct((N, 64, 1), jnp.float32)),
        grid=(N,),
        in_specs=[
            pl.BlockSpec((1, C, P), lambda n: (n, 0, 0)),
            pl.BlockSpec((64, 32), lambda n: (0, 0)),
        ],
        out_specs=(
            pl.BlockSpec((1, 64, 1), lambda n: (n, 0, 0)),
            pl.BlockSpec((1, 64, 1), lambda n: (n, 0, 0)),
        ),
        compiler_params=pltpu.CompilerParams(
            dimension_semantics=("parallel",)),
    )(x2, w_all)

    count = float(N) * 8.0 * float(P)
    s_c = jnp.sum(s_p.reshape(N, 2, 2, 2, _OF), axis=(0, 1, 2, 3))
    q_c = jnp.sum(q_p.reshape(N, 2, 2, 2, _OF), axis=(0, 1, 2, 3))
    mean = s_c / count
    var = q_c / count - mean * mean
    scale = gamma * lax.rsqrt(var + _EPS)
    shift = beta - mean * scale
    scale_rows = jnp.broadcast_to(scale.reshape(1, 1, _OF), (2, 2, _OF)).reshape(32, 1)
    shift_rows = jnp.broadcast_to(shift.reshape(1, 1, _OF), (2, 2, _OF)).reshape(32, 1)
    w2 = w_all.reshape(2, 32, 32)
    x_up = jnp.repeat(x2, 2, axis=-1)

    out = pl.pallas_call(
        functools.partial(_fused_kernel, D=D, H=H, W=W),
        out_shape=jax.ShapeDtypeStruct((N, _OF, D, 2, H, 2, 2 * W),
                                       jnp.float32),
        grid=(N,),
        in_specs=[
            pl.BlockSpec((1, C, 2 * P), lambda n: (n, 0, 0)),
            pl.BlockSpec((32, 32), lambda n: (0, 0)),
            pl.BlockSpec((32, 32), lambda n: (0, 0)),
            pl.BlockSpec((32, 1), lambda n: (0, 0)),
            pl.BlockSpec((32, 1), lambda n: (0, 0)),
        ],
        out_specs=pl.BlockSpec((1, _OF, D, 2, H, 2, 2 * W),
                               lambda n: (n, 0, 0, 0, 0, 0, 0)),
        compiler_params=pltpu.CompilerParams(
            dimension_semantics=("parallel",)),
    )(x_up, w2[0], w2[1], scale_rows, shift_rows)

    return out.reshape(N, _OF, 2 * D, 2 * H, 2 * W)
